# Initial kernel scaffold; baseline (speedup 1.0000x reference)
import jax
import jax.numpy as jnp
from jax.experimental import pallas as pl


def kernel(msa, seq, idx, W_emb, b_emb, emb_q, emb_left, emb_right, pos_emb):
    raise NotImplementedError("write your pallas kernel here")


# speedup vs baseline: 9.9704x; 9.9704x over previous
import functools

import jax
import jax.numpy as jnp
from jax.experimental import pallas as pl
from jax.experimental.pallas import tpu as pltpu

B, N, L = 1, 256, 384
D_INIT, D_MSA, D_PAIR = 46, 256, 128
NBIN = 65
NBIN_PAD = 72
NSEQ_PAD = 32

N_BLK = 8
I_BLK = 16


def _msa_body(seq_ref, msa_ref, wt_ref, b_ref, embq_ref, out_ref, q_scr):
    @pl.when(pl.program_id(0) == 0)
    def _():
        seq = seq_ref[0, :]
        oh = (seq[:, None] == jax.lax.broadcasted_iota(jnp.int32, (L, NSEQ_PAD), 1))
        q = jnp.dot(oh.astype(jnp.float32), embq_ref[...],
                    preferred_element_type=jnp.float32)
        q_scr[...] = q + b_ref[0, :][None, :]

    x = msa_ref[...].reshape(N_BLK * L, D_INIT)
    y = jnp.dot(x, wt_ref[...], preferred_element_type=jnp.float32)
    out_ref[...] = y.reshape(N_BLK, L, D_MSA) + q_scr[...][None, :, :]


def _pair_body(seq_ref, idx_ref, seqb_ref, idxb_ref, el_ref, er_ref, pe_ref,
               out_ref, left_scr):
    i = pl.program_id(0)

    @pl.when(i == 0)
    def _():
        seq = seq_ref[0, :]
        oh = (seq[:, None] == jax.lax.broadcasted_iota(jnp.int32, (L, NSEQ_PAD), 1))
        left_scr[...] = jnp.dot(oh.astype(jnp.float32), el_ref[...],
                                preferred_element_type=jnp.float32)

    seq_i = seqb_ref[0, 0, :]
    oh_r = (seq_i[:, None] == jax.lax.broadcasted_iota(jnp.int32, (I_BLK, NSEQ_PAD), 1))
    right = jnp.dot(oh_r.astype(jnp.float32), er_ref[...],
                    preferred_element_type=jnp.float32)

    idx_j = idx_ref[0, :]
    idx_i = idxb_ref[0, 0, :]
    bucket = jnp.clip(idx_j[None, :] - idx_i[:, None] + 32, 0, NBIN - 1)
    oh_p = (bucket[:, :, None] ==
            jax.lax.broadcasted_iota(jnp.int32, (I_BLK, L, NBIN_PAD), 2))
    pos = jnp.dot(oh_p.astype(jnp.float32).reshape(I_BLK * L, NBIN_PAD), pe_ref[...],
                  preferred_element_type=jnp.float32).reshape(I_BLK, L, D_PAIR)
    out_ref[...] = pos + left_scr[...][None, :, :] + right[:, None, :]


@functools.partial(jax.jit, static_argnames=())
def kernel(msa, seq, idx, W_emb, b_emb, emb_q, emb_left, emb_right, pos_emb):
    msa3 = msa.reshape(N, L, D_INIT)
    seq2 = seq.reshape(1, L).astype(jnp.int32)
    idx2 = idx.reshape(1, L).astype(jnp.int32)
    wt = W_emb.T
    b2 = b_emb.reshape(1, D_MSA)
    embq_p = jnp.zeros((NSEQ_PAD, D_MSA), jnp.float32).at[:22].set(emb_q)
    el_p = jnp.zeros((NSEQ_PAD, D_PAIR), jnp.float32).at[:22].set(emb_left)
    er_p = jnp.zeros((NSEQ_PAD, D_PAIR), jnp.float32).at[:22].set(emb_right)
    pe_p = jnp.zeros((NBIN_PAD, D_PAIR), jnp.float32).at[:NBIN].set(pos_emb)

    msa_e = pl.pallas_call(
        _msa_body,
        grid=(N // N_BLK,),
        in_specs=[
            pl.BlockSpec((1, L), lambda n: (0, 0)),
            pl.BlockSpec((N_BLK, L, D_INIT), lambda n: (n, 0, 0)),
            pl.BlockSpec((D_INIT, D_MSA), lambda n: (0, 0)),
            pl.BlockSpec((1, D_MSA), lambda n: (0, 0)),
            pl.BlockSpec((NSEQ_PAD, D_MSA), lambda n: (0, 0)),
        ],
        out_specs=pl.BlockSpec((N_BLK, L, D_MSA), lambda n: (n, 0, 0)),
        out_shape=jax.ShapeDtypeStruct((N, L, D_MSA), jnp.float32),
        scratch_shapes=[pltpu.VMEM((L, D_MSA), jnp.float32)],
    )(seq2, msa3, wt, b2, embq_p)

    seqb = seq2.reshape(L // I_BLK, 1, I_BLK)
    idxb = idx2.reshape(L // I_BLK, 1, I_BLK)
    pair = pl.pallas_call(
        _pair_body,
        grid=(L // I_BLK,),
        in_specs=[
            pl.BlockSpec((1, L), lambda i: (0, 0)),
            pl.BlockSpec((1, L), lambda i: (0, 0)),
            pl.BlockSpec((1, 1, I_BLK), lambda i: (i, 0, 0)),
            pl.BlockSpec((1, 1, I_BLK), lambda i: (i, 0, 0)),
            pl.BlockSpec((NSEQ_PAD, D_PAIR), lambda i: (0, 0)),
            pl.BlockSpec((NSEQ_PAD, D_PAIR), lambda i: (0, 0)),
            pl.BlockSpec((NBIN_PAD, D_PAIR), lambda i: (0, 0)),
        ],
        out_specs=pl.BlockSpec((I_BLK, L, D_PAIR), lambda i: (i, 0, 0)),
        out_shape=jax.ShapeDtypeStruct((L, L, D_PAIR), jnp.float32),
        scratch_shapes=[pltpu.VMEM((L, D_PAIR), jnp.float32)],
    )(seq2, idx2, seqb, idxb, el_p, er_p, pe_p)

    return (msa_e.reshape(B, N, L, D_MSA), pair.reshape(B, L, L, D_PAIR))
